# baseline (speedup 1.0000x reference)
import functools

import jax
import jax.numpy as jnp
from jax import lax
from jax.experimental import pallas as pl
from jax.experimental.pallas import tpu as pltpu
from jax.experimental.pallas import tpu_sc as plsc

WORD_DIM = 100
POS_DIM = 16
LEM_DIM = 100
OUT_DIM = WORD_DIM + POS_DIM + LEM_DIM
LANE_TILE = 128
POS_VOCAB = 64

CHUNK = 64
PAIR = 2
NTILE = 4


def _sc_embed(word_idx, pos_idx, lem_idx, wtab_p, pos_flat, ltab_p):
    n = word_idx.shape[0]
    info = plsc.get_sparse_core_info()
    nw = info.num_cores * info.num_subcores
    per_w = n // nw
    chunks = per_w // CHUNK
    assert chunks % NTILE == 0 and chunks >= 2 * NTILE
    mesh = plsc.VectorSubcoreMesh(core_axis_name="c", subcore_axis_name="s")

    idx_t = pltpu.VMEM((CHUNK,), jnp.int32)
    lrows_t = pltpu.VMEM((CHUNK, LANE_TILE), jnp.float32)
    tile_t = pltpu.VMEM((CHUNK, OUT_DIM), jnp.float32)
    sem_t = pltpu.SemaphoreType.DMA

    @functools.partial(
        pl.kernel,
        mesh=mesh,
        compiler_params=pltpu.CompilerParams(needs_layout_passes=False),
        out_type=jax.ShapeDtypeStruct((n, OUT_DIM), jnp.float32),
        scratch_types=[
            idx_t, idx_t, idx_t, idx_t, idx_t, idx_t,
            lrows_t, lrows_t,
            tile_t, tile_t, tile_t, tile_t,
            pltpu.VMEM((POS_VOCAB * POS_DIM,), jnp.float32),
            sem_t, sem_t, sem_t, sem_t, sem_t, sem_t,
            sem_t, sem_t, sem_t, sem_t,
        ],
    )
    def k(widx_hbm, pidx_hbm, lidx_hbm, wtab_hbm, pflat_hbm, ltab_hbm,
          out_hbm, wi0, li0, pi0, wi1, li1, pi1, lr0, lr1,
          tile0, tile1, tile2, tile3, pos_v,
          isem0, isem1, psem0, psem1, gsem0, gsem1,
          wsem0, wsem1, wsem2, wsem3):
        wid = lax.axis_index("s") * info.num_cores + lax.axis_index("c")
        w_base = wid * per_w
        lane = lax.iota(jnp.int32, 16)
        wl_idx = ((wi0, li0), (wi1, li1))
        pi_bufs = (pi0, pi1)
        l_bufs = (lr0, lr1)
        tiles = (tile0, tile1, tile2, tile3)
        isems = (isem0, isem1)
        psems = (psem0, psem1)
        gsems = (gsem0, gsem1)
        wsems = (wsem0, wsem1, wsem2, wsem3)

        def base(g):
            return w_base + g * CHUNK

        def fire_idx_wl(g, s):
            b = base(g)
            pltpu.async_copy(widx_hbm.at[pl.ds(b, CHUNK)], wl_idx[s][0],
                             isems[s])
            pltpu.async_copy(lidx_hbm.at[pl.ds(b, CHUNK)], wl_idx[s][1],
                             isems[s])

        def wait_idx_wl(g, s):
            b = base(g)
            pltpu.make_async_copy(widx_hbm.at[pl.ds(b, CHUNK)], wl_idx[s][0],
                                  isems[s]).wait()
            pltpu.make_async_copy(lidx_hbm.at[pl.ds(b, CHUNK)], wl_idx[s][1],
                                  isems[s]).wait()

        def fire_idx_p(g, s):
            pltpu.async_copy(
                pidx_hbm.at[pl.ds(base(g), CHUNK)], pi_bufs[s], psems[s])

        def wait_idx_p(g, s):
            pltpu.make_async_copy(
                pidx_hbm.at[pl.ds(base(g), CHUNK)], pi_bufs[s],
                psems[s]).wait()

        def fire_gathers(s, t):
            pltpu.async_copy(wtab_hbm.at[wl_idx[s][0]],
                             tiles[t].at[:, 0:LANE_TILE], gsems[s])
            pltpu.async_copy(ltab_hbm.at[wl_idx[s][1]], l_bufs[s], gsems[s])

        def wait_gathers(s, t):
            pltpu.make_async_copy(wtab_hbm.at[wl_idx[s][0]],
                                  tiles[t].at[:, 0:LANE_TILE],
                                  gsems[s]).wait()
            pltpu.make_async_copy(ltab_hbm.at[wl_idx[s][1]], l_bufs[s],
                                  gsems[s]).wait()

        def write_copy(g, t):
            return pltpu.make_async_copy(
                tiles[t], out_hbm.at[pl.ds(base(g), CHUNK)], wsems[t])

        def fire_write(g, t):
            pltpu.async_copy(
                tiles[t], out_hbm.at[pl.ds(base(g), CHUNK)], wsems[t])

        def assemble(s, t):
            lr = l_bufs[s]
            pi_v = pi_bufs[s]
            tile_v = tiles[t]

            def pos_row_splat(r):
                return plsc.load_gather(pi_v, [jnp.broadcast_to(r, (16,))])

            def pos_fetch(splat16, colv, mask):
                cv = jnp.where(mask, colv, 0)
                return plsc.load_gather(pos_v, [splat16 * POS_DIM + cv],
                                        mask=mask)

            def lem_fetch(r, colv, mask):
                cv = jnp.where(mask, colv, 0)
                return plsc.load_gather(lr, [jnp.broadcast_to(r, (16,)), cv],
                                        mask=mask)

            def assemble_pair(g, carry):
                r0 = PAIR * g
                r1 = r0 + 1
                p0 = pos_row_splat(r0) * POS_DIM
                p1 = pos_row_splat(r1) * POS_DIM
                m = lane >= 4
                v = plsc.load_gather(
                    pos_v, [p0 + jnp.where(m, lane - 4, 0)], mask=m)
                plsc.store_scatter(tile_v,
                                   [jnp.broadcast_to(r0, (16,)), 96 + lane],
                                   v, mask=m)
                m = lane < 4
                va = plsc.load_gather(pos_v, [p0 + jnp.where(m, lane + 12, 0)],
                                      mask=m)
                vb = lem_fetch(r0, lane - 4, ~m)
                tile_v[r0, pl.ds(112, 16)] = jnp.where(m, va, vb)
                for jj in range(5):
                    tile_v[r0, pl.ds(128 + 16 * jj, 16)] = (
                        lr[r0, pl.ds(12 + 16 * jj, 16)])
                m = lane < 8
                v = lem_fetch(r0, 92 + lane, m)
                plsc.store_scatter(tile_v,
                                   [jnp.broadcast_to(r0, (16,)), 208 + lane],
                                   v, mask=m)
                m = lane >= 12
                v = plsc.load_gather(
                    pos_v, [p1 + jnp.where(m, lane - 12, 0)], mask=m)
                plsc.store_scatter(tile_v,
                                   [jnp.broadcast_to(r1, (16,)), 88 + lane],
                                   v, mask=m)
                m = lane < 12
                va = plsc.load_gather(pos_v, [p1 + jnp.where(m, lane + 4, 0)],
                                      mask=m)
                vb = lem_fetch(r1, lane - 12, ~m)
                tile_v[r1, pl.ds(104, 16)] = jnp.where(m, va, vb)
                v = lr[r1, pl.ds(4, 16)]
                plsc.store_scatter(tile_v,
                                   [jnp.broadcast_to(r1, (16,)), 120 + lane],
                                   v)
                for jj in range(5):
                    tile_v[r1, pl.ds(136 + 16 * jj, 16)] = (
                        lr[r1, pl.ds(20 + 16 * jj, 16)])
                return carry

            lax.fori_loop(0, CHUNK // PAIR, assemble_pair, 0)

        pltpu.sync_copy(pflat_hbm, pos_v)

        fire_idx_wl(0, 0)
        fire_idx_p(0, 0)
        fire_idx_wl(1, 1)
        fire_idx_p(1, 1)
        wait_idx_wl(0, 0)
        fire_gathers(0, 0)

        def chunk_step(g, jmod, m_pos, steady):
            s = jmod % 2
            t = jmod % NTILE
            s1 = (jmod + 1) % 2
            t1 = (jmod + 1) % NTILE
            if steady or g + 1 < chunks:
                wait_idx_wl(g + 1, s1)
                if m_pos is None:
                    write_copy(g + 1 - NTILE, t1).wait()
                else:
                    @pl.when(m_pos)
                    def _():
                        write_copy(g + 1 - NTILE, t1).wait()
                fire_gathers(s1, t1)
            wait_gathers(s, t)
            if steady or g + 2 < chunks:
                fire_idx_wl(g + 2, s)
            wait_idx_p(g, s)
            assemble(s, t)
            if steady or g + 2 < chunks:
                fire_idx_p(g + 2, s)
            fire_write(g, t)

        def steady_body(m, carry):
            G = NTILE * m
            for j in range(NTILE):
                guard = (m > 0) if j < NTILE - 1 else None
                chunk_step(G + j, j, guard, True)
            return carry

        lax.fori_loop(0, chunks // NTILE - 1, steady_body, 0)

        G = chunks - NTILE
        for j in range(NTILE):
            g = G + j
            m_pos = None if g + 1 - NTILE >= 0 else False
            chunk_step(g, j, m_pos, False)
        for j in range(NTILE):
            write_copy(chunks - NTILE + j, j % NTILE).wait()

    return k(word_idx, pos_idx, lem_idx, wtab_p, pos_flat, ltab_p)


def kernel(word_idx, pos_idx, lem_idx, word_table, pos_table, lem_table):
    b, l = word_idx.shape
    n = b * l
    wi = word_idx.reshape(n).astype(jnp.int32)
    pi = pos_idx.reshape(n).astype(jnp.int32)
    li = lem_idx.reshape(n).astype(jnp.int32)
    wtab_p = jnp.pad(word_table, ((0, 0), (0, LANE_TILE - WORD_DIM)))
    ltab_p = jnp.pad(lem_table, ((0, 0), (0, LANE_TILE - LEM_DIM)))
    pos_flat = pos_table.reshape(POS_VOCAB * POS_DIM)
    out = _sc_embed(wi, pi, li, wtab_p, pos_flat, ltab_p)
    return out.reshape(b, l, OUT_DIM)
